# baseline (speedup 1.0000x reference)
import functools

import jax
import jax.numpy as jnp
from jax import lax
from jax.experimental import pallas as pl
from jax.experimental.pallas import tpu as pltpu
from jax.experimental.pallas import tpu_sc as plsc

BATCH = 16384
DIM = 32
ROWS = 1000000
GROW = 128
NTC = (ROWS + GROW - 1) // GROW
NC = 2
NS = 16
NW = NC * NS
BPW = BATCH // NW
WPW = BPW * DIM
CHUNK = 128
WINDOW = 8
TPW = 245


def _pump_body(uwt_hbm, iwt_hbm, uwd_hbm, iwd_hbm, sem_u, sem_i):
    wid = lax.axis_index("s") * NC + lax.axis_index("c")
    start = wid * TPW
    count = jnp.minimum(TPW, jnp.maximum(NTC - start, 0))

    def chunk(k, _):
        tc = start + k
        c0 = pl.multiple_of(tc * GROW, GROW)
        r0 = pl.multiple_of(tc * DIM, 8)
        pltpu.async_copy(uwt_hbm.at[:, pl.ds(c0, GROW)],
                         uwd_hbm.at[pl.ds(r0, DIM), :], sem_u)
        pltpu.async_copy(iwt_hbm.at[:, pl.ds(c0, GROW)],
                         iwd_hbm.at[pl.ds(r0, DIM), :], sem_i)
        @pl.when(k > 0)
        def _wait_prev():
            pltpu.make_async_copy(uwt_hbm.at[:, pl.ds(0, GROW)],
                                  uwd_hbm.at[pl.ds(0, DIM), :], sem_u).wait()
            pltpu.make_async_copy(iwt_hbm.at[:, pl.ds(0, GROW)],
                                  iwd_hbm.at[pl.ds(0, DIM), :], sem_i).wait()
        return _

    lax.fori_loop(0, count, chunk, 0)

    @pl.when(count > 0)
    def _wait_last():
        pltpu.make_async_copy(uwt_hbm.at[:, pl.ds(0, GROW)],
                              uwd_hbm.at[pl.ds(0, DIM), :], sem_u).wait()
        pltpu.make_async_copy(iwt_hbm.at[:, pl.ds(0, GROW)],
                              iwd_hbm.at[pl.ds(0, DIM), :], sem_i).wait()


def _dot_body(u_hbm, i_hbm, uwf_hbm, iwf_hbm, out_hbm,
              uidx_v, iidx_v, uw_idx, iw_idx, ue_v, ie_v, out_v,
              sem_u, sem_i):
    wid = lax.axis_index("s") * NC + lax.axis_index("c")
    base_b = wid * BPW

    pltpu.sync_copy(u_hbm.at[pl.ds(base_b, BPW)], uidx_v)
    pltpu.sync_copy(i_hbm.at[pl.ds(base_b, BPW)], iidx_v)

    def build(g, _):
        b0 = g * 16
        rvec_u = uidx_v[pl.ds(b0, 16)]
        rvec_i = iidx_v[pl.ds(b0, 16)]
        base_u = (rvec_u >> 7) * (DIM * GROW) + (rvec_u & (GROW - 1))
        base_i = (rvec_i >> 7) * (DIM * GROW) + (rvec_i & (GROW - 1))
        for d in range(DIM):
            uw_idx[pl.ds(d * BPW + b0, 16)] = base_u + d * GROW
            iw_idx[pl.ds(d * BPW + b0, 16)] = base_i + d * GROW
        return _

    lax.fori_loop(0, BPW // 16, build, 0)

    pending = []
    for c in range(WPW // CHUNK):
        o = c * CHUNK
        pending.append(pltpu.async_copy(
            uwf_hbm.at[uw_idx.at[pl.ds(o, CHUNK)]],
            ue_v.at[pl.ds(o, CHUNK)], sem_u))
        pending.append(pltpu.async_copy(
            iwf_hbm.at[iw_idx.at[pl.ds(o, CHUNK)]],
            ie_v.at[pl.ds(o, CHUNK)], sem_i))
        while len(pending) > 2 * WINDOW:
            pending.pop(0).wait()
    while pending:
        pending.pop(0).wait()

    def group(h, _):
        c0 = h * 16
        acc = ue_v[pl.ds(c0, 16)] * ie_v[pl.ds(c0, 16)]
        for d in range(1, DIM):
            o = d * BPW + c0
            acc = acc + ue_v[pl.ds(o, 16)] * ie_v[pl.ds(o, 16)]
        out_v[pl.ds(c0, 16)] = acc
        return _

    lax.fori_loop(0, BPW // 16, group, 0)

    pltpu.sync_copy(out_v, out_hbm.at[pl.ds(base_b, BPW)])


@jax.jit
def _mf(u, i, uwt, iwt):
    mesh = plsc.VectorSubcoreMesh(core_axis_name="c", subcore_axis_name="s")
    uwd, iwd = pl.kernel(
        _pump_body,
        out_type=(jax.ShapeDtypeStruct((NTC * DIM, GROW), jnp.float32),
                  jax.ShapeDtypeStruct((NTC * DIM, GROW), jnp.float32)),
        mesh=mesh,
        compiler_params=pltpu.CompilerParams(
            needs_layout_passes=False, use_tc_tiling_on_sc=True,
            disable_bounds_checks=True),
        scratch_types=[
            pltpu.SemaphoreType.DMA,
            pltpu.SemaphoreType.DMA,
        ],
    )(uwt, iwt)

    return pl.kernel(
        _dot_body,
        out_type=jax.ShapeDtypeStruct((BATCH,), jnp.float32),
        mesh=mesh,
        compiler_params=pltpu.CompilerParams(
            needs_layout_passes=False, use_tc_tiling_on_sc=False),
        scratch_types=[
            pltpu.VMEM((BPW,), jnp.int32),
            pltpu.VMEM((BPW,), jnp.int32),
            pltpu.VMEM((WPW,), jnp.int32),
            pltpu.VMEM((WPW,), jnp.int32),
            pltpu.VMEM((WPW,), jnp.float32),
            pltpu.VMEM((WPW,), jnp.float32),
            pltpu.VMEM((BPW,), jnp.float32),
            pltpu.SemaphoreType.DMA,
            pltpu.SemaphoreType.DMA,
        ],
    )(u, i, uwd.reshape(-1), iwd.reshape(-1))


def kernel(u, i, user_weight, item_weight):
    return _mf(u, i, user_weight.T, item_weight.T)


# speedup vs baseline: 31.7722x; 31.7722x over previous
import functools

import jax
import jax.numpy as jnp
from jax import lax
from jax.experimental import pallas as pl
from jax.experimental.pallas import tpu as pltpu
from jax.experimental.pallas import tpu_sc as plsc

BATCH = 16384
DIM = 32
ROWS = 1000000
GROW = 128
NTC = (ROWS + GROW - 1) // GROW
NC = 2
NS = 16
NW = NC * NS
BPW = BATCH // NW
WPW = BPW * DIM
CHUNK = 128
WINDOW = 8
TPW = 245


def _pump_body(uwt_hbm, iwt_hbm, uwd_hbm, iwd_hbm, ubuf_v, ibuf_v,
               sem_ui, sem_uo, sem_ii, sem_io):
    wid = lax.axis_index("s") * NC + lax.axis_index("c")
    start = wid * TPW
    count = jnp.minimum(TPW, jnp.maximum(NTC - start, 0))

    def wait_in(sem, hbm, buf, slot):
        pltpu.make_async_copy(hbm.at[:, pl.ds(0, GROW)], buf.at[slot],
                              sem.at[slot]).wait()

    def wait_out(sem, buf, hbm, slot):
        pltpu.make_async_copy(buf.at[slot], hbm.at[pl.ds(0, DIM), :],
                              sem.at[slot]).wait()

    def emit(kp):
        sp = kp % 4
        rp = pl.multiple_of((start + kp) * DIM, 8)
        wait_in(sem_ui, uwt_hbm, ubuf_v, sp)
        wait_in(sem_ii, iwt_hbm, ibuf_v, sp)
        pltpu.async_copy(ubuf_v.at[sp], uwd_hbm.at[pl.ds(rp, DIM), :],
                         sem_uo.at[sp])
        pltpu.async_copy(ibuf_v.at[sp], iwd_hbm.at[pl.ds(rp, DIM), :],
                         sem_io.at[sp])

    def chunk(k, _):
        s = k % 4

        @pl.when(k >= 4)
        def _reclaim():
            wait_out(sem_uo, ubuf_v, uwd_hbm, s)
            wait_out(sem_io, ibuf_v, iwd_hbm, s)

        tc = start + k
        c0 = pl.multiple_of(tc * GROW, GROW)
        pltpu.async_copy(uwt_hbm.at[:, pl.ds(c0, GROW)], ubuf_v.at[s],
                         sem_ui.at[s])
        pltpu.async_copy(iwt_hbm.at[:, pl.ds(c0, GROW)], ibuf_v.at[s],
                         sem_ii.at[s])

        @pl.when(k >= 2)
        def _emit_prev():
            emit(k - 2)
        return _

    lax.fori_loop(0, count, chunk, 0)

    def tail(k, _):
        @pl.when(k < count)
        def _emit():
            emit(k)
        return _

    lax.fori_loop(jnp.maximum(count - 2, 0), count, tail, 0)

    def drain(k, _):
        @pl.when(k < count)
        def _d():
            wait_out(sem_uo, ubuf_v, uwd_hbm, k % 4)
            wait_out(sem_io, ibuf_v, iwd_hbm, k % 4)
        return _

    lax.fori_loop(jnp.maximum(count - 4, 0), count, drain, 0)


def _dot_body(u_hbm, i_hbm, uwf_hbm, iwf_hbm, out_hbm,
              uidx_v, iidx_v, uw_idx, iw_idx, ue_v, ie_v, out_v,
              sem_u, sem_i):
    wid = lax.axis_index("s") * NC + lax.axis_index("c")
    base_b = wid * BPW

    pltpu.sync_copy(u_hbm.at[pl.ds(base_b, BPW)], uidx_v)
    pltpu.sync_copy(i_hbm.at[pl.ds(base_b, BPW)], iidx_v)

    def build(g, _):
        b0 = g * 16
        rvec_u = uidx_v[pl.ds(b0, 16)]
        rvec_i = iidx_v[pl.ds(b0, 16)]
        base_u = (rvec_u >> 7) * (DIM * GROW) + (rvec_u & (GROW - 1))
        base_i = (rvec_i >> 7) * (DIM * GROW) + (rvec_i & (GROW - 1))
        for d in range(DIM):
            uw_idx[pl.ds(d * BPW + b0, 16)] = base_u + d * GROW
            iw_idx[pl.ds(d * BPW + b0, 16)] = base_i + d * GROW
        return _

    lax.fori_loop(0, BPW // 16, build, 0)

    pending = []
    for c in range(WPW // CHUNK):
        o = c * CHUNK
        pending.append(pltpu.async_copy(
            uwf_hbm.at[uw_idx.at[pl.ds(o, CHUNK)]],
            ue_v.at[pl.ds(o, CHUNK)], sem_u))
        pending.append(pltpu.async_copy(
            iwf_hbm.at[iw_idx.at[pl.ds(o, CHUNK)]],
            ie_v.at[pl.ds(o, CHUNK)], sem_i))
        while len(pending) > 2 * WINDOW:
            pending.pop(0).wait()
    while pending:
        pending.pop(0).wait()

    def group(h, _):
        c0 = h * 16
        acc = ue_v[pl.ds(c0, 16)] * ie_v[pl.ds(c0, 16)]
        for d in range(1, DIM):
            o = d * BPW + c0
            acc = acc + ue_v[pl.ds(o, 16)] * ie_v[pl.ds(o, 16)]
        out_v[pl.ds(c0, 16)] = acc
        return _

    lax.fori_loop(0, BPW // 16, group, 0)

    pltpu.sync_copy(out_v, out_hbm.at[pl.ds(base_b, BPW)])


@jax.jit
def _mf(u, i, uwt, iwt):
    mesh = plsc.VectorSubcoreMesh(core_axis_name="c", subcore_axis_name="s")
    uwd, iwd = pl.kernel(
        _pump_body,
        out_type=(jax.ShapeDtypeStruct((NTC * DIM, GROW), jnp.float32),
                  jax.ShapeDtypeStruct((NTC * DIM, GROW), jnp.float32)),
        mesh=mesh,
        compiler_params=pltpu.CompilerParams(
            needs_layout_passes=False, use_tc_tiling_on_sc=True,
            disable_bounds_checks=True),
        scratch_types=[
            pltpu.VMEM((4, DIM, GROW), jnp.float32),
            pltpu.VMEM((4, DIM, GROW), jnp.float32),
            pltpu.SemaphoreType.DMA((4,)),
            pltpu.SemaphoreType.DMA((4,)),
            pltpu.SemaphoreType.DMA((4,)),
            pltpu.SemaphoreType.DMA((4,)),
        ],
    )(uwt, iwt)

    return pl.kernel(
        _dot_body,
        out_type=jax.ShapeDtypeStruct((BATCH,), jnp.float32),
        mesh=mesh,
        compiler_params=pltpu.CompilerParams(
            needs_layout_passes=False, use_tc_tiling_on_sc=False),
        scratch_types=[
            pltpu.VMEM((BPW,), jnp.int32),
            pltpu.VMEM((BPW,), jnp.int32),
            pltpu.VMEM((WPW,), jnp.int32),
            pltpu.VMEM((WPW,), jnp.int32),
            pltpu.VMEM((WPW,), jnp.float32),
            pltpu.VMEM((WPW,), jnp.float32),
            pltpu.VMEM((BPW,), jnp.float32),
            pltpu.SemaphoreType.DMA,
            pltpu.SemaphoreType.DMA,
        ],
    )(u, i, uwd.reshape(-1), iwd.reshape(-1))


def kernel(u, i, user_weight, item_weight):
    return _mf(u, i, user_weight.T, item_weight.T)
